# Initial kernel scaffold; baseline (speedup 1.0000x reference)
import jax
import jax.numpy as jnp
from jax.experimental import pallas as pl


def kernel(pred, target):
    raise NotImplementedError("write your pallas kernel here")


# speedup vs baseline: 3923.7128x; 3923.7128x over previous
import functools

import jax
import jax.numpy as jnp
from jax.experimental import pallas as pl

_ALPHA = 3.0
_MIN_DIFF = 0.1
_N = 4096

_BR = 256
_BC = 1024


def _mserank_tile(p_col_ref, t_col_ref, p_row_ref, t_row_ref,
                  loss_ref, cnt_ref, reg_ref):
    ri = pl.program_id(0)
    ci = pl.program_id(1)

    @pl.when(jnp.logical_and(ri == 0, ci == 0))
    def _init():
        loss_ref[...] = jnp.zeros((1, 1), jnp.float32)
        cnt_ref[...] = jnp.zeros((1, 1), jnp.float32)
        reg_ref[...] = jnp.zeros((1, 1), jnp.float32)

    p_i = p_col_ref[...]
    t_i = t_col_ref[...]
    p_j = p_row_ref[...]
    t_j = t_row_ref[...]

    d = t_i - t_j
    c = jnp.abs(d)
    x = (p_i - p_j) * jnp.sign(d)
    sp = jnp.maximum(-x, 0.0) + jnp.log1p(jnp.exp(-jnp.abs(x)))
    mask = c > _MIN_DIFF
    loss_ref[...] += jnp.sum(jnp.where(mask, c * sp, 0.0), keepdims=True)
    cnt_ref[...] += jnp.sum(jnp.where(mask, 1.0, 0.0), keepdims=True)

    @pl.when(ci == 0)
    def _reg():
        e = p_i - t_i
        reg_ref[...] += jnp.sum(e * e, keepdims=True)


@jax.jit
def kernel(pred, target):
    p = pred.reshape(_N, 1)
    t = target.reshape(_N, 1)
    p_row = pred.reshape(1, _N)
    t_row = target.reshape(1, _N)

    grid = (_N // _BR, _N // _BC)
    loss_sum, cnt, reg_sum = pl.pallas_call(
        _mserank_tile,
        grid=grid,
        in_specs=[
            pl.BlockSpec((_BR, 1), lambda r, c: (r, 0)),
            pl.BlockSpec((_BR, 1), lambda r, c: (r, 0)),
            pl.BlockSpec((1, _BC), lambda r, c: (0, c)),
            pl.BlockSpec((1, _BC), lambda r, c: (0, c)),
        ],
        out_specs=[
            pl.BlockSpec((1, 1), lambda r, c: (0, 0)),
            pl.BlockSpec((1, 1), lambda r, c: (0, 0)),
            pl.BlockSpec((1, 1), lambda r, c: (0, 0)),
        ],
        out_shape=[
            jax.ShapeDtypeStruct((1, 1), jnp.float32),
            jax.ShapeDtypeStruct((1, 1), jnp.float32),
            jax.ShapeDtypeStruct((1, 1), jnp.float32),
        ],
    )(p, t, p_row, t_row)

    loss_sum = loss_sum[0, 0]
    cnt = cnt[0, 0]
    reg = reg_sum[0, 0] / _N
    pair_mean = loss_sum / jnp.maximum(cnt, 1.0)
    return jnp.where(cnt > 0, reg + _ALPHA * pair_mean, reg)
